# baseline (speedup 1.0000x reference)
import functools

import jax
import jax.numpy as jnp
from jax import lax
from jax.experimental import pallas as pl
from jax.experimental.pallas import tpu as pltpu
from jax.experimental.pallas import tpu_sc as plsc

N_NEG = 50
D = 32
L = 16

_LOG1P = (
    0.9999992249459306,
    -0.4999677773260973,
    0.3328626878693824,
    -0.2465484102008357,
    0.18517671305376252,
    -0.12601773504363445,
    0.0671992182399122,
    -0.023381649402895242,
    0.003824912525210834,
)


def _softplus(x):
    u = jnp.exp(-jnp.abs(x))
    p = jnp.full((L,), _LOG1P[-1], jnp.float32)
    for c in _LOG1P[-2::-1]:
        p = p * u + c
    return jnp.maximum(x, 0.0) + p * u


def _sc_body(tgt_hbm, ctx_hbm, tw_hbm, cw_hbm, neg_hbm, out_hbm,
             tw_v, cw_v, neg_v, trow, crow, nrows, out_v, sem):
    wid = lax.axis_index("c") * 16 + lax.axis_index("s")

    @pl.when(wid == 0)
    def _():
        di0 = pltpu.async_copy(tw_hbm, tw_v, sem)
        di1 = pltpu.async_copy(cw_hbm, cw_v, sem)
        di2 = pltpu.async_copy(neg_hbm, neg_v, sem)
        zero = jnp.zeros((L,), jnp.float32)
        for r in range(N_NEG, 64):
            nrows[r, pl.ds(0, L)] = zero
            nrows[r, pl.ds(L, L)] = zero
        di0.wait()
        di1.wait()
        di2.wait()

        dg0 = pltpu.async_copy(tgt_hbm.at[tw_v], trow, sem)
        dg1 = pltpu.async_copy(ctx_hbm.at[cw_v], crow, sem)
        dg2 = pltpu.async_copy(ctx_hbm.at[neg_v], nrows.at[pl.ds(0, N_NEG)], sem)
        dg0.wait()
        dg1.wait()
        dg2.wait()

        t0 = trow[0, pl.ds(0, L)]
        t1 = trow[0, pl.ds(L, L)]
        c0 = crow[0, pl.ds(0, L)]
        c1 = crow[0, pl.ds(L, L)]
        p_pos = jnp.sum(t0 * c0 + t1 * c1)

        lane = lax.iota(jnp.int32, L)
        t_scalar = [t0[d] for d in range(L)] + [t1[d] for d in range(L)]
        total = _softplus(jnp.where(lane == 0, -p_pos, jnp.float32(-30.0)))
        for chunk in range(4):
            rows = lane + chunk * L
            acc = jnp.zeros((L,), jnp.float32)
            for d in range(D):
                col = jnp.full((L,), d, jnp.int32)
                acc = acc + t_scalar[d] * plsc.load_gather(nrows, [rows, col])
            n_valid = min(max(N_NEG - chunk * L, 0), L)
            x = jnp.where(lane < n_valid, acc, jnp.float32(-30.0))
            total = total + _softplus(x)

        out_v[...] = jnp.zeros((L,), jnp.float32) + jnp.sum(total)
        pltpu.sync_copy(out_v, out_hbm)


@functools.cache
def _build():
    mesh = plsc.VectorSubcoreMesh(core_axis_name="c", subcore_axis_name="s")
    return pl.kernel(
        _sc_body,
        out_type=jax.ShapeDtypeStruct((L,), jnp.float32),
        mesh=mesh,
        scratch_types=[
            pltpu.VMEM((1,), jnp.int32),
            pltpu.VMEM((1,), jnp.int32),
            pltpu.VMEM((N_NEG,), jnp.int32),
            pltpu.VMEM((1, D), jnp.float32),
            pltpu.VMEM((1, D), jnp.float32),
            pltpu.VMEM((64, D), jnp.float32),
            pltpu.VMEM((L,), jnp.float32),
            pltpu.SemaphoreType.DMA,
        ],
        compiler_params=pltpu.CompilerParams(
            needs_layout_passes=False, use_tc_tiling_on_sc=False),
    )


def kernel(embeddings_target, embeddings_context, target_word, context_word,
           negative_context_words):
    out = _build()(embeddings_target, embeddings_context, target_word,
                   context_word, negative_context_words)
    return out[0]


# speedup vs baseline: 1.5038x; 1.5038x over previous
import functools

import jax
import jax.numpy as jnp
from jax import lax
from jax.experimental import pallas as pl
from jax.experimental.pallas import tpu as pltpu
from jax.experimental.pallas import tpu_sc as plsc

N_NEG = 50
D = 32
L = 16

_LOG1P = (
    0.9999992249459306,
    -0.4999677773260973,
    0.3328626878693824,
    -0.2465484102008357,
    0.18517671305376252,
    -0.12601773504363445,
    0.0671992182399122,
    -0.023381649402895242,
    0.003824912525210834,
)


def _softplus(x):
    u = jnp.exp(-jnp.abs(x))
    p = jnp.full((L,), _LOG1P[-1], jnp.float32)
    for c in _LOG1P[-2::-1]:
        p = p * u + c
    return jnp.maximum(x, 0.0) + p * u


def _sc_body(tgt_hbm, ctx_hbm, tw_hbm, cw_hbm, neg_hbm, out_hbm,
             tw_v, cw_v, neg_v, trow, crow, nrows, out_v, sem):
    wid = lax.axis_index("c") * 16 + lax.axis_index("s")

    @pl.when(wid == 0)
    def _():
        di0 = pltpu.async_copy(tw_hbm, tw_v.at[pl.ds(0, 1)], sem)
        di1 = pltpu.async_copy(cw_hbm, cw_v.at[pl.ds(0, 1)], sem)
        di2 = pltpu.async_copy(neg_hbm, neg_v.at[pl.ds(0, N_NEG)], sem)
        zero = jnp.zeros((L,), jnp.float32)
        for r in range(N_NEG, 64):
            nrows[r, pl.ds(0, L)] = zero
            nrows[r, pl.ds(L, L)] = zero
        di0.wait()
        di1.wait()
        di2.wait()

        it = tw_v[pl.ds(0, L)]
        ic = cw_v[pl.ds(0, L)]
        copies = [
            pltpu.async_copy(tgt_hbm.at[pl.ds(it[0], 1)], trow, sem),
            pltpu.async_copy(ctx_hbm.at[pl.ds(ic[0], 1)], crow, sem),
        ]
        for k in range(4):
            nv = neg_v[pl.ds(k * L, L)]
            for j in range(min(N_NEG - k * L, L)):
                r = k * L + j
                copies.append(pltpu.async_copy(
                    ctx_hbm.at[pl.ds(nv[j], 1)], nrows.at[pl.ds(r, 1)], sem))
        for c in copies:
            c.wait()

        t0 = trow[0, pl.ds(0, L)]
        t1 = trow[0, pl.ds(L, L)]
        c0 = crow[0, pl.ds(0, L)]
        c1 = crow[0, pl.ds(L, L)]
        p_pos = jnp.sum(t0 * c0 + t1 * c1)

        lane = lax.iota(jnp.int32, L)
        t_scalar = [t0[d] for d in range(L)] + [t1[d] for d in range(L)]
        total = _softplus(jnp.where(lane == 0, -p_pos, jnp.float32(-30.0)))
        for chunk in range(4):
            rows = lane + chunk * L
            acc = jnp.zeros((L,), jnp.float32)
            for d in range(D):
                col = jnp.full((L,), d, jnp.int32)
                acc = acc + t_scalar[d] * plsc.load_gather(nrows, [rows, col])
            n_valid = min(max(N_NEG - chunk * L, 0), L)
            x = jnp.where(lane < n_valid, acc, jnp.float32(-30.0))
            total = total + _softplus(x)

        out_v[...] = jnp.zeros((L,), jnp.float32) + jnp.sum(total)
        pltpu.sync_copy(out_v, out_hbm)


@functools.cache
def _build():
    mesh = plsc.VectorSubcoreMesh(core_axis_name="c", subcore_axis_name="s")
    return pl.kernel(
        _sc_body,
        out_type=jax.ShapeDtypeStruct((L,), jnp.float32),
        mesh=mesh,
        scratch_types=[
            pltpu.VMEM((L,), jnp.int32),
            pltpu.VMEM((L,), jnp.int32),
            pltpu.VMEM((64,), jnp.int32),
            pltpu.VMEM((1, D), jnp.float32),
            pltpu.VMEM((1, D), jnp.float32),
            pltpu.VMEM((64, D), jnp.float32),
            pltpu.VMEM((L,), jnp.float32),
            pltpu.SemaphoreType.DMA,
        ],
        compiler_params=pltpu.CompilerParams(needs_layout_passes=False),
    )


def kernel(embeddings_target, embeddings_context, target_word, context_word,
           negative_context_words):
    out = _build()(embeddings_target, embeddings_context, target_word,
                   context_word, negative_context_words)
    return out[0]


# speedup vs baseline: 1.5135x; 1.0065x over previous
import functools

import jax
import jax.numpy as jnp
from jax import lax
from jax.experimental import pallas as pl
from jax.experimental.pallas import tpu as pltpu
from jax.experimental.pallas import tpu_sc as plsc

N_NEG = 50
D = 32
L = 16

_LOG1P = (
    0.9999992249459306,
    -0.4999677773260973,
    0.3328626878693824,
    -0.2465484102008357,
    0.18517671305376252,
    -0.12601773504363445,
    0.0671992182399122,
    -0.023381649402895242,
    0.003824912525210834,
)


def _softplus(x):
    u = jnp.exp(-jnp.abs(x))
    p = jnp.full((L,), _LOG1P[-1], jnp.float32)
    for c in _LOG1P[-2::-1]:
        p = p * u + c
    return jnp.maximum(x, 0.0) + p * u


def _fold(v, z):
    for s in (8, 4, 2, 1):
        z[pl.ds(0, L)] = v
        v = v + z[pl.ds(s, L)]
    return v


def _sc_body(tgt_hbm, ctx_hbm, tw_hbm, cw_hbm, neg_hbm, out_hbm,
             tw_v, cw_v, neg_v, trow, crow, nrows, z, out_v, sem):
    wid = lax.axis_index("c") * 16 + lax.axis_index("s")

    @pl.when(wid == 0)
    def _():
        di0 = pltpu.async_copy(tw_hbm, tw_v.at[pl.ds(0, 1)], sem)
        di1 = pltpu.async_copy(cw_hbm, cw_v.at[pl.ds(0, 1)], sem)
        di2 = pltpu.async_copy(neg_hbm, neg_v.at[pl.ds(0, N_NEG)], sem)
        z[pl.ds(L, L)] = jnp.zeros((L,), jnp.float32)
        di0.wait()
        di1.wait()
        di2.wait()

        it = tw_v[pl.ds(0, L)]
        ic = cw_v[pl.ds(0, L)]
        copies = [
            pltpu.async_copy(tgt_hbm.at[pl.ds(it[0], 1)], trow, sem),
            pltpu.async_copy(ctx_hbm.at[pl.ds(ic[0], 1)], crow, sem),
        ]
        for k in range(4):
            nv = neg_v[pl.ds(k * L, L)]
            for j in range(min(N_NEG - k * L, L)):
                r = k * L + j
                copies.append(pltpu.async_copy(
                    ctx_hbm.at[pl.ds(nv[j], 1)], nrows.at[pl.ds(r, 1)], sem))
        for c in copies:
            c.wait()

        t0 = trow[0, pl.ds(0, L)]
        t1 = trow[0, pl.ds(L, L)]
        c0 = crow[0, pl.ds(0, L)]
        c1 = crow[0, pl.ds(L, L)]
        lane = lax.iota(jnp.int32, L)
        p_pos = _fold(t0 * c0 + t1 * c1, z)
        total = _softplus(jnp.where(lane == 0, -p_pos, jnp.float32(-30.0)))

        for r in range(N_NEG):
            n0 = nrows[r, pl.ds(0, L)]
            n1 = nrows[r, pl.ds(L, L)]
            dvec = _fold(n0 * t0 + n1 * t1, z)
            total = total + _softplus(dvec)

        out_v[...] = total
        pltpu.sync_copy(out_v, out_hbm)


@functools.cache
def _build():
    mesh = plsc.VectorSubcoreMesh(core_axis_name="c", subcore_axis_name="s")
    return pl.kernel(
        _sc_body,
        out_type=jax.ShapeDtypeStruct((L,), jnp.float32),
        mesh=mesh,
        scratch_types=[
            pltpu.VMEM((L,), jnp.int32),
            pltpu.VMEM((L,), jnp.int32),
            pltpu.VMEM((64,), jnp.int32),
            pltpu.VMEM((1, D), jnp.float32),
            pltpu.VMEM((1, D), jnp.float32),
            pltpu.VMEM((N_NEG, D), jnp.float32),
            pltpu.VMEM((2 * L,), jnp.float32),
            pltpu.VMEM((L,), jnp.float32),
            pltpu.SemaphoreType.DMA,
        ],
    )


def kernel(embeddings_target, embeddings_context, target_word, context_word,
           negative_context_words):
    out = _build()(embeddings_target, embeddings_context, target_word,
                   context_word, negative_context_words)
    return out[0]


# speedup vs baseline: 37.2928x; 24.6395x over previous
import functools

import jax
import jax.numpy as jnp
from jax import lax
from jax.experimental import pallas as pl
from jax.experimental.pallas import tpu as pltpu
from jax.experimental.pallas import tpu_sc as plsc

N_NEG = 50
D = 32
L = 16
WPT = 4
CTX_SLOT = 56

_LOG1P = (
    0.9999992249459306,
    -0.4999677773260973,
    0.3328626878693824,
    -0.2465484102008357,
    0.18517671305376252,
    -0.12601773504363445,
    0.0671992182399122,
    -0.023381649402895242,
    0.003824912525210834,
)


def _softplus(x):
    u = jnp.exp(-jnp.abs(x))
    p = jnp.full((L,), _LOG1P[-1], jnp.float32)
    for c in _LOG1P[-2::-1]:
        p = p * u + c
    return jnp.maximum(x, 0.0) + p * u


def _fold(v, z):
    for s in (8, 4, 2, 1):
        z[pl.ds(0, L)] = v
        v = v + z[pl.ds(s, L)]
    return v


def _aligned(idx):
    return pl.multiple_of(idx & -128, 128)


def _pick(vec, sub, ebuf):
    ebuf[pl.ds(0, L)] = vec
    return ebuf[pl.ds(sub, L)][0]


def _sc_body(tgt_hbm, ctx_hbm, tw_hbm, cw_hbm, neg_hbm, out_hbm,
             tw_v, widx, tblk, wblks, ebuf, z, pstage, pbuf, out_v, spm, sem):
    core = lax.axis_index("c")
    sid = lax.axis_index("s")

    @pl.when(core == 0)
    def _():
        di0 = pltpu.async_copy(tw_hbm, tw_v.at[pl.ds(0, 1)], sem)
        di1 = pltpu.async_copy(neg_hbm, widx.at[pl.ds(0, N_NEG)], sem)
        di2 = pltpu.async_copy(cw_hbm, widx.at[pl.ds(CTX_SLOT, 1)], sem)
        z[pl.ds(L, L)] = jnp.zeros((L,), jnp.float32)
        di0.wait()
        di1.wait()
        di2.wait()

        @pl.when(sid == 0)
        def _():
            pltpu.sync_copy(z.at[pl.ds(L, L)], spm)

        it0 = tw_v[pl.ds(0, L)][0]
        wv = widx[pl.ds(sid * WPT, L)]
        slot0 = sid * WPT
        raw = [wv[j] for j in range(WPT)]
        gids = [slot0 + j for j in range(WPT)]
        valid = [(g < N_NEG) | (g == CTX_SLOT) for g in gids]
        idxs = [jnp.where(valid[j], raw[j], 0) for j in range(WPT)]

        copies = [pltpu.async_copy(
            tgt_hbm.at[:, pl.ds(_aligned(it0), 128)], tblk, sem)]
        for j in range(WPT):
            copies.append(pltpu.async_copy(
                ctx_hbm.at[:, pl.ds(_aligned(idxs[j]), 128)],
                wblks.at[j], sem))
        for c in copies:
            c.wait()

        t_off = it0 & 112
        t_sub = it0 & 15
        t_sc = [_pick(tblk[d, pl.ds(t_off, L)], t_sub, ebuf) for d in range(D)]

        lane = lax.iota(jnp.int32, L)
        xv = jnp.full((L,), -30.0, jnp.float32)
        for j in range(WPT):
            off = idxs[j] & 112
            sub = idxs[j] & 15
            acc = jnp.zeros((L,), jnp.float32)
            for d in range(D):
                acc = acc + t_sc[d] * wblks[j, d, pl.ds(off, L)]
            dot = _pick(acc, sub, ebuf)
            g = gids[j]
            x = jnp.where(g == CTX_SLOT, -dot,
                          jnp.where(g < N_NEG, dot, jnp.float32(-30.0)))
            xv = jnp.where(lane == j, x, xv)

        pstage[...] = _softplus(xv)
        plsc.subcore_barrier()
        pltpu.sync_copy(pstage, spm.at[lane], add=True)
        plsc.subcore_barrier()

        @pl.when(sid == 0)
        def _():
            pltpu.sync_copy(spm, pbuf)
            out_v[...] = _fold(pbuf[pl.ds(0, L)], z)
            pltpu.sync_copy(out_v, out_hbm)


@functools.cache
def _build():
    mesh = plsc.VectorSubcoreMesh(core_axis_name="c", subcore_axis_name="s")
    return pl.kernel(
        _sc_body,
        out_type=jax.ShapeDtypeStruct((L,), jnp.float32),
        mesh=mesh,
        scratch_types=[
            pltpu.VMEM((L,), jnp.int32),
            pltpu.VMEM((5 * L,), jnp.int32),
            pltpu.VMEM((D, 128), jnp.float32),
            pltpu.VMEM((WPT, D, 128), jnp.float32),
            pltpu.VMEM((2 * L,), jnp.float32),
            pltpu.VMEM((2 * L,), jnp.float32),
            pltpu.VMEM((L,), jnp.float32),
            pltpu.VMEM((L,), jnp.float32),
            pltpu.VMEM((L,), jnp.float32),
            pltpu.VMEM_SHARED((L,), jnp.float32),
            pltpu.SemaphoreType.DMA,
        ],
    )


def kernel(embeddings_target, embeddings_context, target_word, context_word,
           negative_context_words):
    out = _build()(embeddings_target.T, embeddings_context.T, target_word,
                   context_word, negative_context_words)
    return out[0]


# speedup vs baseline: 39.5704x; 1.0611x over previous
import functools

import jax
import jax.numpy as jnp
from jax import lax
from jax.experimental import pallas as pl
from jax.experimental.pallas import tpu as pltpu
from jax.experimental.pallas import tpu_sc as plsc

N_NEG = 50
D = 32
L = 16
WPT = 4
CTX_SLOT = 56

_LOG1P = (
    0.9999992249459306,
    -0.4999677773260973,
    0.3328626878693824,
    -0.2465484102008357,
    0.18517671305376252,
    -0.12601773504363445,
    0.0671992182399122,
    -0.023381649402895242,
    0.003824912525210834,
)


def _softplus(x):
    u = jnp.exp(-jnp.abs(x))
    p = jnp.full((L,), _LOG1P[-1], jnp.float32)
    for c in _LOG1P[-2::-1]:
        p = p * u + c
    return jnp.maximum(x, 0.0) + p * u


def _fold(v, z):
    for s in (8, 4, 2, 1):
        z[pl.ds(0, L)] = v
        v = v + z[pl.ds(s, L)]
    return v


def _aligned(idx):
    return pl.multiple_of(idx & -128, 128)


def _pick(vec, sub, ebuf):
    ebuf[pl.ds(0, L)] = vec
    return ebuf[pl.ds(sub, L)][0]


def _sc_body(tgt_hbm, ctx_hbm, tw_hbm, cw_hbm, neg_hbm, out_hbm,
             tw_v, widx, tblk, wblks, ebuf, z, pstage, pbuf, out_v, spm, sem):
    core = lax.axis_index("c")
    sid = lax.axis_index("s")

    @pl.when(core == 0)
    def _():
        di0 = pltpu.async_copy(tw_hbm, tw_v.at[pl.ds(0, 1)], sem)
        di1 = pltpu.async_copy(neg_hbm, widx.at[pl.ds(0, N_NEG)], sem)
        di2 = pltpu.async_copy(cw_hbm, widx.at[pl.ds(CTX_SLOT, 1)], sem)
        z[pl.ds(L, L)] = jnp.zeros((L,), jnp.float32)
        di0.wait()
        di1.wait()
        di2.wait()

        @pl.when(sid == 0)
        def _():
            pltpu.sync_copy(z.at[pl.ds(L, L)], spm)

        it0 = tw_v[pl.ds(0, L)][0]
        wv = widx[pl.ds(sid * WPT, L)]
        slot0 = sid * WPT
        raw = [wv[j] for j in range(WPT)]
        gids = [slot0 + j for j in range(WPT)]
        valid = [(g < N_NEG) | (g == CTX_SLOT) for g in gids]
        idxs = [jnp.where(valid[j], raw[j], 0) for j in range(WPT)]

        copies = [pltpu.async_copy(
            tgt_hbm.at[:, pl.ds(_aligned(it0), 128)], tblk, sem)]
        for j in range(WPT):
            copies.append(pltpu.async_copy(
                ctx_hbm.at[:, pl.ds(_aligned(idxs[j]), 128)],
                wblks.at[j], sem))
        for c in copies:
            c.wait()

        t_off = it0 & 112
        t_sub = it0 & 15
        t_sc = [_pick(tblk[d, pl.ds(t_off, L)], t_sub, ebuf) for d in range(D)]

        lane = lax.iota(jnp.int32, L)
        xv = jnp.full((L,), -30.0, jnp.float32)
        for j in range(WPT):
            off = idxs[j] & 112
            sub = idxs[j] & 15
            acc = jnp.zeros((L,), jnp.float32)
            for d in range(D):
                acc = acc + t_sc[d] * wblks[j, d, pl.ds(off, L)]
            dot = _pick(acc, sub, ebuf)
            g = gids[j]
            x = jnp.where(g == CTX_SLOT, -dot,
                          jnp.where(g < N_NEG, dot, jnp.float32(-30.0)))
            xv = jnp.where(lane == j, x, xv)

        pstage[...] = _softplus(xv)
        plsc.subcore_barrier()
        pltpu.sync_copy(pstage, spm.at[lane], add=True)
        plsc.subcore_barrier()

        @pl.when(sid == 0)
        def _():
            pltpu.sync_copy(spm, pbuf)
            out_v[...] = _fold(pbuf[pl.ds(0, L)], z)
            pltpu.sync_copy(out_v, out_hbm)


@functools.cache
def _build():
    mesh = plsc.VectorSubcoreMesh(core_axis_name="c", subcore_axis_name="s",
                                  num_cores=1)
    return pl.kernel(
        _sc_body,
        out_type=jax.ShapeDtypeStruct((L,), jnp.float32),
        mesh=mesh,
        scratch_types=[
            pltpu.VMEM((L,), jnp.int32),
            pltpu.VMEM((5 * L,), jnp.int32),
            pltpu.VMEM((D, 128), jnp.float32),
            pltpu.VMEM((WPT, D, 128), jnp.float32),
            pltpu.VMEM((2 * L,), jnp.float32),
            pltpu.VMEM((2 * L,), jnp.float32),
            pltpu.VMEM((L,), jnp.float32),
            pltpu.VMEM((L,), jnp.float32),
            pltpu.VMEM((L,), jnp.float32),
            pltpu.VMEM_SHARED((L,), jnp.float32),
            pltpu.SemaphoreType.DMA,
        ],
        compiler_params=pltpu.CompilerParams(
            disable_bounds_checks=True,
            disable_semaphore_checks=True,
            skip_device_barrier=True,
        ),
    )


def kernel(embeddings_target, embeddings_context, target_word, context_word,
           negative_context_words):
    out = _build()(embeddings_target.T, embeddings_context.T, target_word,
                   context_word, negative_context_words)
    return out[0]
